# Initial kernel scaffold; baseline (speedup 1.0000x reference)
import jax
import jax.numpy as jnp
from jax.experimental import pallas as pl


def kernel(x, table):
    raise NotImplementedError("write your pallas kernel here")


# speedup vs baseline: 4.0860x; 4.0860x over previous
import functools

import jax
import jax.numpy as jnp
from jax import lax
from jax.experimental import pallas as pl
from jax.experimental.pallas import tpu as pltpu
from jax.experimental.pallas import tpu_sc as plsc

VOCAB = 100000
DIM = 64
CHUNK = 128


def _make_gather(n_total: int):
  info = plsc.get_sparse_core_info()
  nc, ns = info.num_cores, info.num_subcores
  nw = nc * ns
  assert n_total % (nw * CHUNK) == 0
  steps = n_total // (nw * CHUNK)
  b_per_w = steps * CHUNK

  mesh = plsc.VectorSubcoreMesh(core_axis_name="c", subcore_axis_name="s")

  @functools.partial(
      pl.kernel,
      mesh=mesh,
      compiler_params=pltpu.CompilerParams(use_tc_tiling_on_sc=False),
      out_type=jax.ShapeDtypeStruct((n_total, DIM), jnp.float32),
      scratch_types=[
          pltpu.VMEM((steps, CHUNK), jnp.int32),
          pltpu.VMEM((CHUNK, DIM), jnp.float32),
          pltpu.SemaphoreType.DMA,
      ],
  )
  def gather_kernel(idx_hbm, table_hbm, out_hbm, idx_v, rows_v, sem):
    wid = lax.axis_index("s") * nc + lax.axis_index("c")
    out_base = wid * b_per_w
    pltpu.sync_copy(idx_hbm.at[wid], idx_v)

    def step(i, carry):
      pltpu.async_copy(table_hbm.at[idx_v.at[i]], rows_v, sem).wait()
      pltpu.sync_copy(rows_v, out_hbm.at[pl.ds(out_base + i * CHUNK, CHUNK)])
      return carry

    lax.fori_loop(0, steps, step, 0)

  return gather_kernel


def kernel(x, table):
  b, s = x.shape
  n_total = b * s
  nw = 32
  idx3d = x.reshape(nw, n_total // (nw * CHUNK), CHUNK)
  out = _make_gather(n_total)(idx3d, table)
  return out.reshape(b, s, DIM)


# speedup vs baseline: 4.6641x; 1.1415x over previous
import functools

import jax
import jax.numpy as jnp
from jax import lax
from jax.experimental import pallas as pl
from jax.experimental.pallas import tpu as pltpu
from jax.experimental.pallas import tpu_sc as plsc

VOCAB = 100000
DIM = 64
CHUNK = 128
NBUF = 8


def _make_gather(n_total: int):
  info = plsc.get_sparse_core_info()
  nc, ns = info.num_cores, info.num_subcores
  nw = nc * ns
  assert n_total % (nw * CHUNK) == 0
  steps = n_total // (nw * CHUNK)
  b_per_w = steps * CHUNK

  mesh = plsc.VectorSubcoreMesh(core_axis_name="c", subcore_axis_name="s")

  @functools.partial(
      pl.kernel,
      mesh=mesh,
      compiler_params=pltpu.CompilerParams(use_tc_tiling_on_sc=False),
      out_type=jax.ShapeDtypeStruct((n_total, DIM), jnp.float32),
      scratch_types=[
          pltpu.VMEM((steps, CHUNK), jnp.int32),
          pltpu.VMEM((NBUF, CHUNK, DIM), jnp.float32),
          pltpu.SemaphoreType.DMA,
      ],
  )
  def gather_kernel(idx_hbm, table_hbm, out_hbm, idx_v, rows_v, gsem):
    wid = lax.axis_index("s") * nc + lax.axis_index("c")
    out_base = wid * b_per_w
    pltpu.sync_copy(idx_hbm.at[wid], idx_v)

    for b in range(NBUF):
      pltpu.async_copy(table_hbm.at[idx_v.at[b]], rows_v.at[b], gsem)

    def drain_store(i):
      b = lax.rem(i, NBUF)
      pltpu.make_async_copy(table_hbm.at[idx_v.at[i]], rows_v.at[b], gsem).wait()
      pltpu.sync_copy(rows_v.at[b], out_hbm.at[pl.ds(out_base + i * CHUNK, CHUNK)])
      return b

    def step_main(i, carry):
      b = drain_store(i)
      pltpu.async_copy(table_hbm.at[idx_v.at[i + NBUF]], rows_v.at[b], gsem)
      return carry

    def step_tail(i, carry):
      drain_store(i)
      return carry

    lax.fori_loop(0, steps - NBUF, step_main, 0)
    lax.fori_loop(steps - NBUF, steps, step_tail, 0)

  return gather_kernel


def kernel(x, table):
  b, s = x.shape
  n_total = b * s
  nw = 32
  idx3d = x.reshape(nw, n_total // (nw * CHUNK), CHUNK)
  out = _make_gather(n_total)(idx3d, table)
  return out.reshape(b, s, DIM)
